# Initial kernel scaffold; baseline (speedup 1.0000x reference)
import jax
import jax.numpy as jnp
from jax.experimental import pallas as pl


def kernel(density_data, links, rand_cells):
    raise NotImplementedError("write your pallas kernel here")


# speedup vs baseline: 35.9210x; 35.9210x over previous
import jax
import jax.numpy as jnp
from jax import lax
from jax.experimental import pallas as pl
from jax.experimental.pallas import tpu as pltpu
from jax.experimental.pallas import tpu_sc as plsc

RES = 256
RES3 = RES * RES * RES
NV = 2000000
HALF = NV // 2
LAMBDA_TV = 1e-05
NC = 2
NS = 16
NW = NC * NS
L = 16
PIECE = 128
CH = 384
K = CH // PIECE
NCH = 28
CPT = CH * NCH
PADN = NS * CPT
ZB = 4096
SZ = 16 * ZB
ACCN = NS * SZ
DUMMY = HALF
TAILN = HALF - (NS - 1) * SZ
TAILF = TAILN // ZB
TAILR = TAILN - TAILF * ZB


def _tv_body(dens_hbm, links_hbm, cells_hbm, out_hbm,
             acc, zbuf, cells_v, i100_v, i010_v, i001_v,
             l000_v, l100_v, l010_v, l001_v,
             s000_v, s100_v, s010_v, s001_v,
             g000_v, g100_v, g010_v, g001_v, sem):
    core = lax.axis_index("c")
    sub = lax.axis_index("s")
    obase = core * HALF

    def _zero(i, carry):
        zbuf[pl.ds(i * L, L)] = jnp.zeros((L,), jnp.float32)
        return carry
    lax.fori_loop(0, ZB // L, _zero, 0)

    def _zcopy(j, carry):
        pltpu.sync_copy(zbuf, acc.at[pl.ds(sub * SZ + j * ZB, ZB)])
        return carry
    lax.fori_loop(0, SZ // ZB, _zcopy, 0)
    plsc.subcore_barrier()

    cbase = sub * CPT

    def _chunk(ci, carry):
        pltpu.sync_copy(cells_hbm.at[pl.ds(cbase + ci * CH, CH)], cells_v)

        def _idx(g, c2):
            sl = pl.ds(g * L, L)
            c = cells_v[sl]
            x = c >> 16
            y = (c >> 8) & 255
            z = c & 255
            i100_v[sl] = c + jnp.where(x < RES - 1, 65536, 0)
            i010_v[sl] = c + jnp.where(y < RES - 1, 256, 0)
            i001_v[sl] = c + jnp.where(z < RES - 1, 1, 0)
            return c2
        lax.fori_loop(0, CH // L, _idx, 0)

        for j in range(K):
            dsl = pl.ds(j * PIECE, PIECE)
            c0 = pltpu.async_copy(links_hbm.at[cells_v.at[dsl]],
                                  l000_v.at[dsl], sem)
            c1 = pltpu.async_copy(links_hbm.at[i100_v.at[dsl]],
                                  l100_v.at[dsl], sem)
            c2 = pltpu.async_copy(links_hbm.at[i010_v.at[dsl]],
                                  l010_v.at[dsl], sem)
            c3 = pltpu.async_copy(links_hbm.at[i001_v.at[dsl]],
                                  l001_v.at[dsl], sem)
            c0.wait(); c1.wait(); c2.wait(); c3.wait()
        for j in range(K):
            dsl = pl.ds(j * PIECE, PIECE)
            c0 = pltpu.async_copy(dens_hbm.at[l000_v.at[dsl]],
                                  g000_v.at[dsl], sem)
            c1 = pltpu.async_copy(dens_hbm.at[l100_v.at[dsl]],
                                  g100_v.at[dsl], sem)
            c2 = pltpu.async_copy(dens_hbm.at[l010_v.at[dsl]],
                                  g010_v.at[dsl], sem)
            c3 = pltpu.async_copy(dens_hbm.at[l001_v.at[dsl]],
                                  g001_v.at[dsl], sem)
            c0.wait(); c1.wait(); c2.wait(); c3.wait()

        def _compute(g, c2_):
            sl = pl.ds(g * L, L)
            row = g // (PIECE // L)
            lane = (g % (PIECE // L)) * L
            c = cells_v[sl]
            x = c >> 16
            y = (c >> 8) & 255
            z = c & 255
            m = (x < RES - 1) & (y < RES - 1) & (z < RES - 1)
            v000 = g000_v[sl]
            v100 = g100_v[sl]
            v010 = g010_v[sl]
            v001 = g001_v[sl]
            dx = v100 - v000
            dy = v010 - v000
            dz = v001 - v000
            ss = 1e-9 + dx * dx + dy * dy + dz * dz
            xi = plsc.bitcast(ss, jnp.int32)
            r = plsc.bitcast(jnp.int32(0x5F3759DF) - (xi >> 1), jnp.float32)
            r = r * (1.5 - 0.5 * ss * r * r)
            r = r * (1.5 - 0.5 * ss * r * r)
            r = r * (1.5 - 0.5 * ss * r * r)
            idelta = jnp.where(m, jnp.float32(LAMBDA_TV), jnp.float32(0.0)) * r
            g000_v[sl] = -(dx + dy + dz) * idelta
            g100_v[sl] = dx * idelta
            g010_v[sl] = dy * idelta
            g001_v[sl] = dz * idelta
            for lv, sv in ((l000_v, s000_v), (l100_v, s100_v),
                           (l010_v, s010_v), (l001_v, s001_v)):
                lk = lv[sl]
                own = (lk >= obase) & (lk < obase + HALF)
                sv[row, pl.ds(lane, L)] = jnp.where(own, lk - obase, DUMMY)
            return c2_
        lax.fori_loop(0, CH // L, _compute, 0)

        for j in range(K):
            dsl = pl.ds(j * PIECE, PIECE)
            c0 = pltpu.async_copy(g000_v.at[dsl], acc.at[s000_v.at[j]],
                                  sem, add=True)
            c1 = pltpu.async_copy(g100_v.at[dsl], acc.at[s100_v.at[j]],
                                  sem, add=True)
            c2 = pltpu.async_copy(g010_v.at[dsl], acc.at[s010_v.at[j]],
                                  sem, add=True)
            c3 = pltpu.async_copy(g001_v.at[dsl], acc.at[s001_v.at[j]],
                                  sem, add=True)
            c0.wait(); c1.wait(); c2.wait(); c3.wait()
        return carry
    lax.fori_loop(0, NCH, _chunk, 0)

    plsc.subcore_barrier()

    def _rcopy(j, carry):
        off = sub * SZ + j * ZB
        pltpu.sync_copy(acc.at[pl.ds(off, ZB)], zbuf)
        pltpu.sync_copy(zbuf, out_hbm.at[pl.ds(obase + off, ZB)])
        return carry

    @pl.when(sub < NS - 1)
    def _():
        lax.fori_loop(0, SZ // ZB, _rcopy, 0)

    @pl.when(sub == NS - 1)
    def _():
        lax.fori_loop(0, TAILF, _rcopy, 0)
        off = sub * SZ + TAILF * ZB
        pltpu.sync_copy(acc.at[pl.ds(off, TAILR)], zbuf.at[pl.ds(0, TAILR)])
        pltpu.sync_copy(zbuf.at[pl.ds(0, TAILR)],
                        out_hbm.at[pl.ds(obase + off, TAILR)])


def kernel(density_data, links, rand_cells):
    dens = density_data.reshape(-1)
    links_f = links.reshape(-1)
    pad = PADN - rand_cells.shape[0]
    cells = jnp.concatenate([
        rand_cells.astype(jnp.int32),
        jnp.full((pad,), RES3 - 1, jnp.int32),
    ])
    mesh = plsc.VectorSubcoreMesh(core_axis_name="c", subcore_axis_name="s")
    out = pl.kernel(
        _tv_body,
        out_type=jax.ShapeDtypeStruct((NV,), jnp.float32),
        mesh=mesh,
        compiler_params=pltpu.CompilerParams(needs_layout_passes=False),
        scratch_types=[
            pltpu.VMEM_SHARED((ACCN,), jnp.float32),
            pltpu.VMEM((ZB,), jnp.float32),
            pltpu.VMEM((CH,), jnp.int32),
            pltpu.VMEM((CH,), jnp.int32),
            pltpu.VMEM((CH,), jnp.int32),
            pltpu.VMEM((CH,), jnp.int32),
            pltpu.VMEM((CH,), jnp.int32),
            pltpu.VMEM((CH,), jnp.int32),
            pltpu.VMEM((CH,), jnp.int32),
            pltpu.VMEM((CH,), jnp.int32),
            pltpu.VMEM((K, PIECE), jnp.int32),
            pltpu.VMEM((K, PIECE), jnp.int32),
            pltpu.VMEM((K, PIECE), jnp.int32),
            pltpu.VMEM((K, PIECE), jnp.int32),
            pltpu.VMEM((CH,), jnp.float32),
            pltpu.VMEM((CH,), jnp.float32),
            pltpu.VMEM((CH,), jnp.float32),
            pltpu.VMEM((CH,), jnp.float32),
            pltpu.SemaphoreType.DMA,
        ],
    )(dens, links_f, cells)
    return out.reshape(NV, 1)
